# baseline (speedup 1.0000x reference)
import jax
import jax.numpy as jnp
from jax import lax
from jax.experimental import pallas as pl
from jax.experimental.pallas import tpu as pltpu

_NTOK = 16384
_HIDDEN = 2048
_NE = 64
_NPAD = 128
_ROWS = 256
_NBUF = 8
_NGRP = _NTOK // (_ROWS * _NBUF)


def _body(b_ref, x_hbm, w_ref, logits_hbm, probs_hbm, cap_hbm,
          xbuf, lbuf, pbuf, cbuf, in_sems, out_sems):
    b = b_ref[0]
    w = w_ref[...]

    def in_copy(part, slot):
        return pltpu.make_async_copy(
            x_hbm.at[pl.ds(part * _ROWS, _ROWS), :],
            xbuf.at[slot],
            in_sems.at[slot],
        )

    def out_copies(part, slot):
        rows = pl.ds(part * _ROWS, _ROWS)
        return (
            pltpu.make_async_copy(lbuf.at[slot], logits_hbm.at[rows, :],
                                  out_sems.at[slot]),
            pltpu.make_async_copy(pbuf.at[slot], probs_hbm.at[rows, :],
                                  out_sems.at[slot]),
            pltpu.make_async_copy(cbuf.at[slot], cap_hbm.at[rows, :],
                                  out_sems.at[slot]),
        )

    for s in range(_NBUF):
        in_copy(s, s).start()

    def group(g, carry):
        for s in range(_NBUF):
            part = g * _NBUF + s
            in_copy(part, s).wait()

            @pl.when(g > 0)
            def _():
                for c in out_copies(part - _NBUF, s):
                    c.wait()

            acc = jnp.dot(xbuf[s], w, preferred_element_type=jnp.float32)
            logits = acc[:, :_NE]
            lbuf[s] = logits
            m = jnp.max(logits, axis=-1, keepdims=True)
            e = jnp.exp(logits - m)
            pbuf[s] = e / jnp.sum(e, axis=-1, keepdims=True)
            cbuf[s] = jax.nn.sigmoid(acc[:, _NE:_NE + 1] + b)

            for c in out_copies(part, s):
                c.start()

            @pl.when(g + 1 < _NGRP)
            def _():
                in_copy(part + _NBUF, s).start()
        return carry

    lax.fori_loop(0, _NGRP, group, 0)

    for s in range(_NBUF):
        for c in out_copies((_NGRP - 1) * _NBUF + s, s):
            c.wait()


def kernel(x, W_router, W_cap, b_cap):
    w_all = jnp.concatenate([W_router, W_cap], axis=0)
    w_all = jnp.pad(w_all, ((0, _NPAD - _NE - 1), (0, 0))).T

    out_shapes = (
        jax.ShapeDtypeStruct((_NTOK, _NE), jnp.float32),
        jax.ShapeDtypeStruct((_NTOK, _NE), jnp.float32),
        jax.ShapeDtypeStruct((_NTOK, 1), jnp.float32),
    )
    hbm = pl.BlockSpec(memory_space=pltpu.MemorySpace.HBM)
    return pl.pallas_call(
        _body,
        grid_spec=pltpu.PrefetchScalarGridSpec(
            num_scalar_prefetch=1,
            grid=(1,),
            in_specs=[
                hbm,
                pl.BlockSpec((_HIDDEN, _NPAD), lambda i, b: (0, 0)),
            ],
            out_specs=[hbm, hbm, hbm],
            scratch_shapes=[
                pltpu.VMEM((_NBUF, _ROWS, _HIDDEN), jnp.float32),
                pltpu.VMEM((_NBUF, _ROWS, _NE), jnp.float32),
                pltpu.VMEM((_NBUF, _ROWS, _NE), jnp.float32),
                pltpu.VMEM((_NBUF, _ROWS, 1), jnp.float32),
                pltpu.SemaphoreType.DMA((_NBUF,)),
                pltpu.SemaphoreType.DMA((_NBUF,)),
            ],
        ),
        out_shape=out_shapes,
        compiler_params=pltpu.CompilerParams(
            dimension_semantics=("arbitrary",),
        ),
    )(b_cap, x, w_all)


# speedup vs baseline: 1.0505x; 1.0505x over previous
import jax
import jax.numpy as jnp
from jax import lax
from jax.experimental import pallas as pl
from jax.experimental.pallas import tpu as pltpu

_NTOK = 16384
_HIDDEN = 2048
_NE = 64
_NPAD = 128
_ROWS = 512
_NBUF = 8
_NGRP = _NTOK // (_ROWS * _NBUF)


def _body(b_ref, x_hbm, w_ref, logits_hbm, probs_hbm, cap_hbm,
          xbuf, lbuf, pbuf, cbuf, in_sems, out_sems):
    b = b_ref[0]
    w = w_ref[...]

    def in_copy(part, slot):
        return pltpu.make_async_copy(
            x_hbm.at[pl.ds(part * _ROWS, _ROWS), :],
            xbuf.at[slot],
            in_sems.at[slot],
        )

    def out_copies(part, slot):
        rows = pl.ds(part * _ROWS, _ROWS)
        return (
            pltpu.make_async_copy(lbuf.at[slot], logits_hbm.at[rows, :],
                                  out_sems.at[slot]),
            pltpu.make_async_copy(pbuf.at[slot], probs_hbm.at[rows, :],
                                  out_sems.at[slot]),
            pltpu.make_async_copy(cbuf.at[slot], cap_hbm.at[rows, :],
                                  out_sems.at[slot]),
        )

    for s in range(_NBUF):
        in_copy(s, s).start()

    def group(g, carry):
        for s in range(_NBUF):
            part = g * _NBUF + s
            in_copy(part, s).wait()

            @pl.when(g > 0)
            def _():
                for c in out_copies(part - _NBUF, s):
                    c.wait()

            acc = jnp.dot(xbuf[s], w, preferred_element_type=jnp.float32)
            logits = acc[:, :_NE]
            lbuf[s] = logits
            m = jnp.max(logits, axis=-1, keepdims=True)
            e = jnp.exp(logits - m)
            pbuf[s] = e / jnp.sum(e, axis=-1, keepdims=True)
            cbuf[s] = jax.nn.sigmoid(acc[:, _NE:_NE + 1] + b)

            for c in out_copies(part, s):
                c.start()

            @pl.when(g + 1 < _NGRP)
            def _():
                in_copy(part + _NBUF, s).start()
        return carry

    lax.fori_loop(0, _NGRP, group, 0)

    for s in range(_NBUF):
        for c in out_copies((_NGRP - 1) * _NBUF + s, s):
            c.wait()


def kernel(x, W_router, W_cap, b_cap):
    w_all = jnp.concatenate([W_router, W_cap], axis=0)
    w_all = jnp.pad(w_all, ((0, _NPAD - _NE - 1), (0, 0))).T

    out_shapes = (
        jax.ShapeDtypeStruct((_NTOK, _NE), jnp.float32),
        jax.ShapeDtypeStruct((_NTOK, _NE), jnp.float32),
        jax.ShapeDtypeStruct((_NTOK, 1), jnp.float32),
    )
    hbm = pl.BlockSpec(memory_space=pltpu.MemorySpace.HBM)
    return pl.pallas_call(
        _body,
        grid_spec=pltpu.PrefetchScalarGridSpec(
            num_scalar_prefetch=1,
            grid=(1,),
            in_specs=[
                hbm,
                pl.BlockSpec((_HIDDEN, _NPAD), lambda i, b: (0, 0)),
            ],
            out_specs=[hbm, hbm, hbm],
            scratch_shapes=[
                pltpu.VMEM((_NBUF, _ROWS, _HIDDEN), jnp.float32),
                pltpu.VMEM((_NBUF, _ROWS, _NE), jnp.float32),
                pltpu.VMEM((_NBUF, _ROWS, _NE), jnp.float32),
                pltpu.VMEM((_NBUF, _ROWS, 1), jnp.float32),
                pltpu.SemaphoreType.DMA((_NBUF,)),
                pltpu.SemaphoreType.DMA((_NBUF,)),
            ],
        ),
        out_shape=out_shapes,
        compiler_params=pltpu.CompilerParams(
            dimension_semantics=("arbitrary",),
        ),
    )(b_cap, x, w_all)


# speedup vs baseline: 1.3532x; 1.2882x over previous
import jax
import jax.numpy as jnp
from jax.experimental import pallas as pl
from jax.experimental.pallas import tpu as pltpu

_NTOK = 16384
_HIDDEN = 2048
_NE = 64
_NPAD = 128
_BLK_M = 2048
_NSPLIT = 8
_PART = _BLK_M // _NSPLIT


def _head(acc, b, logits_ref, probs_ref, cap_ref, rows):
    logits = acc[:, :_NE]
    logits_ref[rows, :] = logits
    m = jnp.max(logits, axis=-1, keepdims=True)
    e = jnp.exp(logits - m)
    probs_ref[rows, :] = e / jnp.sum(e, axis=-1, keepdims=True)
    cap_ref[rows, :] = jax.nn.sigmoid(acc[:, _NE:_NE + 1] + b)


def _body(b_ref, *refs):
    x_refs = refs[:_NSPLIT]
    w_ref, logits_ref, probs_ref, cap_ref = refs[_NSPLIT:]
    w = w_ref[...]
    b = b_ref[0]
    for p in range(_NSPLIT):
        acc = jnp.dot(x_refs[p][...], w, preferred_element_type=jnp.float32)
        _head(acc, b, logits_ref, probs_ref, cap_ref, pl.ds(p * _PART, _PART))


def kernel(x, W_router, W_cap, b_cap):
    w_all = jnp.concatenate([W_router, W_cap], axis=0)
    w_all = jnp.pad(w_all, ((0, _NPAD - _NE - 1), (0, 0))).T

    grid = (_NTOK // _BLK_M,)
    out_shapes = (
        jax.ShapeDtypeStruct((_NTOK, _NE), jnp.float32),
        jax.ShapeDtypeStruct((_NTOK, _NE), jnp.float32),
        jax.ShapeDtypeStruct((_NTOK, 1), jnp.float32),
    )

    def part_spec(p):
        return pl.BlockSpec(
            (_PART, _HIDDEN),
            lambda i, b, p=p: (_NSPLIT * i + p, 0),
        )

    return pl.pallas_call(
        _body,
        grid_spec=pltpu.PrefetchScalarGridSpec(
            num_scalar_prefetch=1,
            grid=grid,
            in_specs=[part_spec(p) for p in range(_NSPLIT)] + [
                pl.BlockSpec((_HIDDEN, _NPAD), lambda i, b: (0, 0)),
            ],
            out_specs=[
                pl.BlockSpec((_BLK_M, _NE), lambda i, b: (i, 0)),
                pl.BlockSpec((_BLK_M, _NE), lambda i, b: (i, 0)),
                pl.BlockSpec((_BLK_M, 1), lambda i, b: (i, 0)),
            ],
        ),
        out_shape=out_shapes,
        compiler_params=pltpu.CompilerParams(
            dimension_semantics=("arbitrary",),
            vmem_limit_bytes=120 * 1024 * 1024,
        ),
    )(b_cap, *([x] * _NSPLIT), w_all)


# speedup vs baseline: 1.3579x; 1.0035x over previous
import jax
import jax.numpy as jnp
from jax.experimental import pallas as pl
from jax.experimental.pallas import tpu as pltpu

_NTOK = 16384
_HIDDEN = 2048
_NE = 64
_NPAD = 128
_BLK_M = 2048
_NSPLIT = 8
_PART = _BLK_M // _NSPLIT


def _head(acc, b, logits_ref, probs_ref, cap_ref, rows):
    logits = acc[:, :_NE]
    logits_ref[rows, :] = logits
    m = jnp.max(logits, axis=-1, keepdims=True)
    e = jnp.exp(logits - m)
    probs_ref[rows, :] = e / jnp.sum(e, axis=-1, keepdims=True)
    cap_ref[rows, :] = jax.nn.sigmoid(acc[:, _NE:_NE + 1] + b)


def _body(b_ref, *refs):
    x_refs = refs[:_NSPLIT]
    w_ref, logits_ref, probs_ref, cap_ref = refs[_NSPLIT:]
    w = w_ref[...]
    b = b_ref[0]
    for p in range(_NSPLIT):
        acc = jnp.dot(x_refs[p][...], w, preferred_element_type=jnp.float32)
        _head(acc, b, logits_ref, probs_ref, cap_ref, pl.ds(p * _PART, _PART))


def kernel(x, W_router, W_cap, b_cap):
    w_all = jnp.concatenate([W_router, W_cap], axis=0)
    w_all = jnp.pad(w_all, ((0, _NPAD - _NE - 1), (0, 0))).T

    grid = (_NTOK // _BLK_M,)
    out_shapes = (
        jax.ShapeDtypeStruct((_NTOK, _NE), jnp.float32),
        jax.ShapeDtypeStruct((_NTOK, _NE), jnp.float32),
        jax.ShapeDtypeStruct((_NTOK, 1), jnp.float32),
    )

    def part_spec(p):
        return pl.BlockSpec(
            (_PART, _HIDDEN),
            lambda i, b, p=p: (_NSPLIT * i + p, 0),
        )

    return pl.pallas_call(
        _body,
        grid_spec=pltpu.PrefetchScalarGridSpec(
            num_scalar_prefetch=1,
            grid=grid,
            in_specs=[part_spec(p) for p in range(_NSPLIT)] + [
                pl.BlockSpec((_HIDDEN, _NPAD), lambda i, b: (0, 0)),
            ],
            out_specs=[
                pl.BlockSpec((_BLK_M, _NE), lambda i, b: (i, 0)),
                pl.BlockSpec((_BLK_M, _NE), lambda i, b: (i, 0)),
                pl.BlockSpec((_BLK_M, 1), lambda i, b: (i, 0)),
            ],
        ),
        out_shape=out_shapes,
        compiler_params=pltpu.CompilerParams(
            dimension_semantics=("arbitrary",),
        ),
    )(b_cap, *([x] * _NSPLIT), w_all)
